# baseline (speedup 1.0000x reference)
import functools

import jax
import jax.numpy as jnp
from jax import lax
from jax.experimental import pallas as pl
from jax.experimental.pallas import tpu as pltpu
from jax.experimental.pallas import tpu_sc as plsc

_D = 128
_P = 32
_NC = 2
_NS = 16
_NW = _NC * _NS
_SPLITS = ((0, 128), (128, 72))


def _emb_combine(idx, pos, tab):
    b_total, s_len = idx.shape
    b_per_w = b_total // _NW
    mesh = plsc.VectorSubcoreMesh(core_axis_name="c", subcore_axis_name="s")

    @functools.partial(
        pl.kernel,
        out_type=jax.ShapeDtypeStruct((b_total, s_len, _D + _P), jnp.float32),
        mesh=mesh,
        scratch_types=[
            pltpu.VMEM((b_per_w, s_len), jnp.int32),
            pltpu.VMEM((s_len, _D), jnp.float32),
            pltpu.VMEM((s_len, _D), jnp.float32),
            pltpu.VMEM((s_len, _P), jnp.float32),
            pltpu.VMEM((s_len, _P), jnp.float32),
            pltpu.SemaphoreType.DMA,
            pltpu.SemaphoreType.DMA,
            pltpu.SemaphoreType.DMA,
            pltpu.SemaphoreType.DMA,
            pltpu.SemaphoreType.DMA,
            pltpu.SemaphoreType.DMA,
        ],
    )
    def body(idx_hbm, pos_hbm, tab_hbm, out_hbm, idx_all,
             word_v0, word_v1, pos_v0, pos_v1,
             sg0, sg1, sp0, sp1, sw0, sw1):
        word_v = (word_v0, word_v1)
        pos_v = (pos_v0, pos_v1)
        sg = (sg0, sg1)
        sp = (sp0, sp1)
        sw = (sw0, sw1)

        wid = lax.axis_index("s") * _NC + lax.axis_index("c")
        b0 = wid * b_per_w

        def start_inputs(i, slot):
            for off, ln in _SPLITS:
                pltpu.async_copy(
                    tab_hbm.at[idx_all.at[i, pl.ds(off, ln)]],
                    word_v[slot].at[pl.ds(off, ln)], sg[slot])
            pltpu.async_copy(pos_hbm.at[b0 + i], pos_v[slot], sp[slot])

        def wait_inputs(slot):
            for off, ln in _SPLITS:
                pltpu.make_async_copy(
                    tab_hbm.at[idx_all.at[0, pl.ds(off, ln)]],
                    word_v[slot].at[pl.ds(off, ln)], sg[slot]).wait()
            pltpu.make_async_copy(pos_hbm.at[0], pos_v[slot], sp[slot]).wait()

        def start_writes(i, slot):
            pltpu.async_copy(
                word_v[slot],
                out_hbm.at[b0 + i, pl.ds(0, s_len), pl.ds(0, _D)], sw[slot])
            pltpu.async_copy(
                pos_v[slot],
                out_hbm.at[b0 + i, pl.ds(0, s_len), pl.ds(_D, _P)], sw[slot])

        def wait_writes(slot):
            pltpu.make_async_copy(
                word_v[slot],
                out_hbm.at[0, pl.ds(0, s_len), pl.ds(0, _D)], sw[slot]).wait()
            pltpu.make_async_copy(
                pos_v[slot],
                out_hbm.at[0, pl.ds(0, s_len), pl.ds(_D, _P)], sw[slot]).wait()

        def step(i, slot, first=False, last=False):
            if not first:
                wait_writes(1 - slot)
            if not last:
                start_inputs(i + 1, 1 - slot)
            wait_inputs(slot)
            start_writes(i, slot)

        pltpu.sync_copy(idx_hbm.at[pl.ds(b0, b_per_w)], idx_all)

        start_inputs(0, 0)
        step(0, 0, first=True)
        step(1, 1)

        def pair(j, carry):
            step(2 * j, 0)
            step(2 * j + 1, 1)
            return carry

        lax.fori_loop(1, b_per_w // 2 - 1, pair, 0)

        step(b_per_w - 2, 0)
        step(b_per_w - 1, 1, last=True)
        wait_writes(1)

    return body(idx, pos, tab)


def kernel(token_ids, pos_onehot, W):
    return _emb_combine(token_ids.astype(jnp.int32),
                        pos_onehot.astype(jnp.float32), W)
